# baseline (speedup 1.0000x reference)
import functools

import jax
import jax.numpy as jnp
from jax import lax
from jax.experimental import pallas as pl
from jax.experimental.pallas import tpu as pltpu, tpu_sc as plsc

N = 50000
E = 800000
G = 512
NAI = 4
NAF = 64
NGF = 2
NRBF = 10
NCONV = 3
NH = 64

HALF = N // 2
PAIRS = HALF // 2
ACCROWS = 12800
TPW = ACCROWS // 16
B = 64
NWIN = E // B
NTILES = 16


def _sc_body(t_hbm, src_hbm, dst_hbm, f_hbm, out_hbm,
             acc_sh, src_v, dst_v, par_v, trows_v, frows_v, msg_v, sem):
    c = lax.axis_index("c")
    s = lax.axis_index("s")
    base_node = c * HALF

    def zrow(r, carry):
        for j in range(8):
            msg_v[r, pl.ds(j * 16, 16)] = jnp.zeros((16,), jnp.float32)
        return carry
    lax.fori_loop(0, B, zrow, 0)
    stripe = pl.multiple_of(s * TPW, TPW)
    for k in range(TPW // B):
        pltpu.sync_copy(msg_v, acc_sh.at[pl.ds(stripe + k * B, B)])
    if TPW % B:
        pltpu.sync_copy(msg_v.at[pl.ds(0, TPW % B)],
                        acc_sh.at[pl.ds(stripe + (TPW // B) * B, TPW % B)])
    plsc.subcore_barrier()

    wlo = (s * NWIN) // NTILES
    whi = ((s + 1) * NWIN) // NTILES

    def win(w, carry):
        ebase = pl.multiple_of(w * B, B)
        fbase = pl.multiple_of(w * (B // 2), B // 2)
        pltpu.sync_copy(src_hbm.at[pl.ds(ebase, B)], src_v)
        pltpu.sync_copy(dst_hbm.at[pl.ds(ebase, B)], dst_v)
        pltpu.sync_copy(f_hbm.at[pl.ds(fbase, B // 2)], frows_v)
        pltpu.async_copy(t_hbm.at[src_v], trows_v, sem).wait()

        lane = lax.iota(jnp.int32, 16)
        for g in range(B // 16):
            d = dst_v[pl.ds(g * 16, 16)]
            local = d - base_node
            ok = (local >= 0) & (local < HALF)
            dump = PAIRS + (g % 8) * 16 + lane
            row = lax.shift_right_arithmetic(local, 1)
            dst_v[pl.ds(g * 16, 16)] = jnp.where(ok, row, dump)
            par_v[pl.ds(g * 16, 16)] = lax.bitwise_and(d, 1)

        zero16 = jnp.zeros((16,), jnp.float32)

        def mrow(rp, carry2):
            pv = par_v[pl.ds(2 * rp, 16)]
            for sub in range(2):
                r = rp * 2 + sub
                off = pv[sub] * 64
                zoff = 64 - off
                for j in range(4):
                    tj = trows_v[r, pl.ds(j * 16, 16)]
                    fj = frows_v[rp, pl.ds(sub * 64 + j * 16, 16)]
                    msg_v[r, pl.ds(off + j * 16, 16)] = tj * fj
                    msg_v[r, pl.ds(zoff + j * 16, 16)] = zero16
            return carry2
        lax.fori_loop(0, B // 2, mrow, 0)

        pltpu.sync_copy(msg_v, acc_sh.at[dst_v], add=True)
        return carry
    lax.fori_loop(wlo, whi, win, 0)
    plsc.subcore_barrier()

    pltpu.sync_copy(acc_sh.at[pl.ds(stripe, TPW)],
                    out_hbm.at[c, pl.ds(stripe, TPW)])


@functools.lru_cache(maxsize=None)
def _sc_msg_pass_fn():
    return pl.kernel(
        _sc_body,
        out_type=jax.ShapeDtypeStruct((2, ACCROWS, 128), jnp.float32),
        mesh=plsc.VectorSubcoreMesh(core_axis_name="c", subcore_axis_name="s",
                                    num_cores=2, num_subcores=NTILES),
        scratch_types=[
            pltpu.VMEM_SHARED((ACCROWS, 128), jnp.float32),
            pltpu.VMEM((B,), jnp.int32),
            pltpu.VMEM((B,), jnp.int32),
            pltpu.VMEM((B + 16,), jnp.int32),
            pltpu.VMEM((B, 128), jnp.float32),
            pltpu.VMEM((B // 2, 128), jnp.float32),
            pltpu.VMEM((B, 128), jnp.float32),
            pltpu.SemaphoreType.DMA,
        ],
    )


_NB = 2000


def _mm_body(x_ref, w_ref, b_ref, o_ref):
    o_ref[...] = jnp.dot(x_ref[...], w_ref[...],
                         preferred_element_type=jnp.float32) + b_ref[...]


def _mm_pad_body(x_ref, w_ref, b_ref, o_ref):
    t = jnp.dot(x_ref[...], w_ref[...],
                preferred_element_type=jnp.float32) + b_ref[...]
    o_ref[...] = jnp.concatenate(
        [t, jnp.zeros((_NB, NAF), jnp.float32)], axis=1)


def _node_matmul(x, w, b, pad=False):
    m, k = x.shape
    n = w.shape[1]
    return pl.pallas_call(
        _mm_pad_body if pad else _mm_body,
        grid=(m // _NB,),
        in_specs=[
            pl.BlockSpec((_NB, k), lambda i: (i, 0)),
            pl.BlockSpec((k, n), lambda i: (0, 0)),
            pl.BlockSpec((1, n), lambda i: (0, 0)),
        ],
        out_specs=pl.BlockSpec((_NB, 2 * n if pad else n), lambda i: (i, 0)),
        out_shape=jax.ShapeDtypeStruct((m, 2 * n if pad else n), jnp.float32),
    )(x, w, b)


_FW = 2000

_RBF_OFF = [6.0 / (NRBF - 1) * k for k in range(NRBF)]
_RBF_COEFF = -0.5 / (_RBF_OFF[1] - _RBF_OFF[0]) ** 2


def _filter_body(d_ref, w_ref, b_ref, o_ref):
    d = d_ref[...]
    acc0 = jnp.zeros((_FW, NAF), jnp.float32) + b_ref[...]
    acc1 = acc0
    for k in range(NRBF):
        eak = jnp.exp(_RBF_COEFF * (d - _RBF_OFF[k]) ** 2)
        acc0 = acc0 + eak[:, 0:1] * w_ref[k][None, :]
        acc1 = acc1 + eak[:, 1:2] * w_ref[k][None, :]
    o_ref[...] = jnp.concatenate([acc0, acc1], axis=1)


def _edge_filter(ea_two, w, b):
    return pl.pallas_call(
        _filter_body,
        grid=(E // 2 // _FW,),
        in_specs=[
            pl.BlockSpec((_FW, 2), lambda i: (i, 0)),
            pl.BlockSpec((NRBF, NAF), lambda i: (0, 0)),
            pl.BlockSpec((1, NAF), lambda i: (0, 0)),
        ],
        out_specs=pl.BlockSpec((_FW, 128), lambda i: (i, 0)),
        out_shape=jax.ShapeDtypeStruct((E // 2, 128), jnp.float32),
    )(ea_two, w, b)


_HB = 1000


def _hupd_body(h_ref, agg_ref, w_ref, b_ref, o_ref):
    z = h_ref[...] + jnp.dot(agg_ref[0], w_ref[...],
                             preferred_element_type=jnp.float32) + b_ref[...]
    o_ref[...] = jax.nn.softplus(z)


def _h_update(h, agg2, w, b):
    return pl.pallas_call(
        _hupd_body,
        grid=(N // _HB,),
        in_specs=[
            pl.BlockSpec((_HB, NAF), lambda i: (i, 0)),
            pl.BlockSpec((1, _HB, NAF),
                         lambda i: (i // (HALF // _HB), i % (HALF // _HB), 0)),
            pl.BlockSpec((NAF, NAF), lambda i: (0, 0)),
            pl.BlockSpec((1, NAF), lambda i: (0, 0)),
        ],
        out_specs=pl.BlockSpec((_HB, NAF), lambda i: (i, 0)),
        out_shape=jax.ShapeDtypeStruct((N, NAF), jnp.float32),
    )(h, agg2, w, b)


def _pool_body(h_ref, batch_ref, u_ref, fcw_ref, fcb_ref,
               hbw1_ref, hbb1_ref, hbw2_ref, hbb2_ref,
               hew1_ref, heb1_ref, hew2_ref, heb2_ref,
               obg_ref, oeh_ref, acc_ref):
    b = pl.program_id(0)

    @pl.when(b == 0)
    def _():
        acc_ref[...] = jnp.zeros((G, 128), jnp.float32)

    gids = lax.broadcasted_iota(jnp.int32, (G, _HB), 0)
    oh = (gids == batch_ref[0, 0, :][None, :]).astype(jnp.float32)
    h_aug = jnp.concatenate(
        [h_ref[...], jnp.ones((_HB, NAF), jnp.float32)], axis=1)
    acc_ref[...] += jnp.dot(oh, h_aug, preferred_element_type=jnp.float32)

    @pl.when(b == N // _HB - 1)
    def _():
        sums = acc_ref[:, :NAF]
        cnt = jnp.maximum(acc_ref[:, NAF:NAF + 1], 1.0)
        c = sums / cnt
        gmp = jnp.concatenate([c, u_ref[...]], axis=1)
        z = jax.nn.relu(jnp.dot(gmp, fcw_ref[...],
                                preferred_element_type=jnp.float32)
                        + fcb_ref[...])
        zb = jax.nn.relu(jnp.dot(z, hbw1_ref[...],
                                 preferred_element_type=jnp.float32)
                         + hbb1_ref[...])
        bg = jnp.dot(zb, hbw2_ref[...],
                     preferred_element_type=jnp.float32) + hbb2_ref[...]
        obg_ref[...] = jnp.log1p(jnp.clip(bg, 0.0, None))
        ze = jax.nn.relu(jnp.dot(z, hew1_ref[...],
                                 preferred_element_type=jnp.float32)
                         + heb1_ref[...])
        oeh_ref[...] = jnp.dot(ze, hew2_ref[...],
                               preferred_element_type=jnp.float32) + heb2_ref[...]


def _pool_heads(h, batch3, u, fc_W, fc_b, hb_W1, hb_b1, hb_W2, hb_b2,
                he_W1, he_b1, he_W2, he_b2):
    nb = N // _HB
    full = lambda shape: pl.BlockSpec(shape, lambda i: tuple(0 for _ in shape))
    return pl.pallas_call(
        _pool_body,
        grid=(nb,),
        in_specs=[
            pl.BlockSpec((_HB, NAF), lambda i: (i, 0)),
            pl.BlockSpec((1, 1, _HB), lambda i: (i, 0, 0)),
            full((G, NGF)),
            full((NAF + NGF, 2 * NH)), full((1, 2 * NH)),
            full((2 * NH, NH)), full((1, NH)), full((NH, 1)), full((1, 1)),
            full((2 * NH, NH)), full((1, NH)), full((NH, 1)), full((1, 1)),
        ],
        out_specs=[full((G, 1)), full((G, 1))],
        out_shape=[jax.ShapeDtypeStruct((G, 1), jnp.float32),
                   jax.ShapeDtypeStruct((G, 1), jnp.float32)],
        scratch_shapes=[pltpu.VMEM((G, 128), jnp.float32)],
    )(h, batch3, u, fc_W, fc_b, hb_W1, hb_b1, hb_W2, hb_b2,
      he_W1, he_b1, he_W2, he_b2)


def kernel(x, edge_index, edge_attr, batch, u,
           emb_W, emb_b, b1_W, b1_b, be_W, be_b, b2_W, b2_b,
           fc_W, fc_b, hb_W1, hb_b1, hb_W2, hb_b2,
           he_W1, he_b1, he_W2, he_b2):
    src = edge_index[0]
    dst = edge_index[1]
    ea_two = edge_attr.reshape(E // 2, 2)

    h = _node_matmul(x, emb_W, emb_b.reshape(1, NAF))
    for i in range(NCONV):
        f = _edge_filter(ea_two, be_W[i], be_b[i].reshape(1, NAF))
        t = _node_matmul(h, b1_W[i], b1_b[i].reshape(1, NAF), pad=True)
        agg2 = _sc_msg_pass_fn()(t, src, dst, f)
        agg2 = agg2.reshape(2, 2 * ACCROWS, NAF)
        h = _h_update(h, agg2, b2_W[i], b2_b[i].reshape(1, NAF))

    out_bg, out_eh = _pool_heads(
        h, batch.reshape(N // _HB, 1, _HB), u,
        fc_W, fc_b.reshape(1, 2 * NH),
        hb_W1, hb_b1.reshape(1, NH), hb_W2, hb_b2.reshape(1, 1),
        he_W1, he_b1.reshape(1, NH), he_W2, he_b2.reshape(1, 1))
    return (out_bg, out_eh)


# speedup vs baseline: 1.5415x; 1.5415x over previous
import functools

import jax
import jax.numpy as jnp
from jax import lax
from jax.experimental import pallas as pl
from jax.experimental.pallas import tpu as pltpu, tpu_sc as plsc

N = 50000
E = 800000
G = 512
NAI = 4
NAF = 64
NGF = 2
NRBF = 10
NCONV = 3
NH = 64

HALF = N // 2
PAIRS = HALF // 2
ACCROWS = 12544
TPW = ACCROWS // 16
B = 64
NWIN = E // B
NTILES = 16


def _sc_body(t_hbm, src_hbm, dst_hbm, f_hbm, out_hbm,
             acc_sh, src2, row2, par2, rowlist_v, trows2, frows_v, msg_v,
             sem_g, sem_f, sem_i):
    c = lax.axis_index("c")
    s = lax.axis_index("s")
    chigh = c > 0

    def zrow(r, carry):
        for j in range(8):
            msg_v[r, pl.ds(j * 16, 16)] = jnp.zeros((16,), jnp.float32)
        return carry
    lax.fori_loop(0, B, zrow, 0)
    stripe = pl.multiple_of(s * TPW, TPW)
    for k in range(TPW // B):
        pltpu.sync_copy(msg_v, acc_sh.at[pl.ds(stripe + k * B, B)])
    if TPW % B:
        pltpu.sync_copy(msg_v.at[pl.ds(0, TPW % B)],
                        acc_sh.at[pl.ds(stripe + (TPW // B) * B, TPW % B)])
    plsc.subcore_barrier()

    wlo = (s * NWIN) // NTILES
    whi = ((s + 1) * NWIN) // NTILES

    def load_idx(w, buf):
        eb = pl.multiple_of(w * B, B)
        pltpu.async_copy(src_hbm.at[pl.ds(eb, B)], src2.at[buf], sem_i)
        pltpu.async_copy(dst_hbm.at[pl.ds(eb, B)], row2.at[buf], sem_i)

    def wait_idx(buf):
        pltpu.make_async_copy(src_hbm.at[pl.ds(0, B)], src2.at[buf],
                              sem_i).wait()
        pltpu.make_async_copy(dst_hbm.at[pl.ds(0, B)], row2.at[buf],
                              sem_i).wait()

    def transform(buf):
        for g in range(B // 16):
            d = row2[buf, pl.ds(g * 16, 16)]
            inhigh = d >= HALF
            ok = jnp.equal(jnp.where(inhigh, 1, 0), c)
            local = jnp.where(inhigh, d - HALF, d)
            row = lax.shift_right_arithmetic(local, 1)
            code = jnp.where(ok, 1 + lax.bitwise_and(d, 1), 0)
            row2[buf, pl.ds(g * 16, 16)] = row
            par2[buf, pl.ds(g * 16, 16)] = code

    def fire_gather(buf):
        pltpu.async_copy(t_hbm.at[src2.at[buf]], trows2.at[buf],
                         sem_g.at[buf])

    def fire_f(w):
        fb = pl.multiple_of(w * (B // 2), B // 2)
        pltpu.async_copy(f_hbm.at[pl.ds(fb, B // 2)], frows_v, sem_f)

    load_idx(wlo, 0)
    wait_idx(0)
    transform(0)
    fire_gather(0)
    fire_f(wlo)
    load_idx(wlo + 1, 1)

    def win(w, carry):
        cur = lax.bitwise_and(w - wlo, 1)
        nxt = 1 - cur

        @pl.when(w + 1 < whi)
        def _():
            wait_idx(nxt)
            transform(nxt)
            fire_gather(nxt)

        pltpu.make_async_copy(t_hbm.at[src2.at[cur]], trows2.at[cur],
                              sem_g.at[cur]).wait()
        pltpu.make_async_copy(f_hbm.at[pl.ds(0, B // 2)], frows_v,
                              sem_f).wait()

        for g in range(B // 16):
            rowlist_v[pl.ds(g * 16, 16)] = row2[cur, pl.ds(g * 16, 16)]

        def mrow(rp, carry2):
            pv = par2[cur, pl.ds(2 * rp, 16)]
            for sub in range(2):
                r = rp * 2 + sub
                code = pv[sub]
                ind0 = jnp.where(code == 1, 1.0, 0.0)
                ind1 = jnp.where(code == 2, 1.0, 0.0)
                for j in range(4):
                    tj = trows2[cur, r, pl.ds(j * 16, 16)]
                    fj = frows_v[rp, pl.ds(sub * 64 + j * 16, 16)]
                    p = tj * fj
                    msg_v[r, pl.ds(j * 16, 16)] = p * ind0
                    msg_v[r, pl.ds(64 + j * 16, 16)] = p * ind1
            return carry2
        lax.fori_loop(0, B // 2, mrow, 0)

        pltpu.sync_copy(msg_v, acc_sh.at[rowlist_v], add=True)

        @pl.when(w + 1 < whi)
        def _():
            fire_f(w + 1)

        @pl.when(w + 2 < whi)
        def _():
            load_idx(w + 2, cur)
        return carry
    lax.fori_loop(wlo, whi, win, 0)
    plsc.subcore_barrier()

    pltpu.sync_copy(acc_sh.at[pl.ds(stripe, TPW)],
                    out_hbm.at[c, pl.ds(stripe, TPW)])


@functools.lru_cache(maxsize=None)
def _sc_msg_pass_fn():
    return pl.kernel(
        _sc_body,
        out_type=jax.ShapeDtypeStruct((2, ACCROWS, 128), jnp.float32),
        mesh=plsc.VectorSubcoreMesh(core_axis_name="c", subcore_axis_name="s",
                                    num_cores=2, num_subcores=NTILES),
        scratch_types=[
            pltpu.VMEM_SHARED((ACCROWS, 128), jnp.float32),
            pltpu.VMEM((2, B), jnp.int32),
            pltpu.VMEM((2, B), jnp.int32),
            pltpu.VMEM((2, B + 16), jnp.int32),
            pltpu.VMEM((B,), jnp.int32),
            pltpu.VMEM((2, B, 128), jnp.float32),
            pltpu.VMEM((B // 2, 128), jnp.float32),
            pltpu.VMEM((B, 128), jnp.float32),
            pltpu.SemaphoreType.DMA((2,)),
            pltpu.SemaphoreType.DMA,
            pltpu.SemaphoreType.DMA,
        ],
    )


_NB = 2000


def _mm_body(x_ref, w_ref, b_ref, o_ref):
    o_ref[...] = jnp.dot(x_ref[...], w_ref[...],
                         preferred_element_type=jnp.float32) + b_ref[...]


def _mm_pad_body(x_ref, w_ref, b_ref, o_ref):
    t = jnp.dot(x_ref[...], w_ref[...],
                preferred_element_type=jnp.float32) + b_ref[...]
    o_ref[...] = jnp.concatenate(
        [t, jnp.zeros((_NB, NAF), jnp.float32)], axis=1)


def _node_matmul(x, w, b, pad=False):
    m, k = x.shape
    n = w.shape[1]
    return pl.pallas_call(
        _mm_pad_body if pad else _mm_body,
        grid=(m // _NB,),
        in_specs=[
            pl.BlockSpec((_NB, k), lambda i: (i, 0)),
            pl.BlockSpec((k, n), lambda i: (0, 0)),
            pl.BlockSpec((1, n), lambda i: (0, 0)),
        ],
        out_specs=pl.BlockSpec((_NB, 2 * n if pad else n), lambda i: (i, 0)),
        out_shape=jax.ShapeDtypeStruct((m, 2 * n if pad else n), jnp.float32),
    )(x, w, b)


_FW = 2000


def _filter_body(d_ref, offs_ref, coeff_ref, w_ref, b_ref, o_ref):
    d = d_ref[...]
    co = coeff_ref[...]
    offs = offs_ref[...]
    halves = []
    for s_ in range(2):
        diff = d[:, s_:s_ + 1] - offs
        ea = jnp.exp(co * diff ** 2)
        halves.append(jnp.dot(ea, w_ref[...],
                              preferred_element_type=jnp.float32)
                      + b_ref[...])
    o_ref[...] = jnp.concatenate(halves, axis=1)


def _edge_filter(ea_two, offs, coeff, w, b):
    return pl.pallas_call(
        _filter_body,
        grid=(E // 2 // _FW,),
        in_specs=[
            pl.BlockSpec((_FW, 2), lambda i: (i, 0)),
            pl.BlockSpec((1, NRBF), lambda i: (0, 0)),
            pl.BlockSpec((1, 1), lambda i: (0, 0)),
            pl.BlockSpec((NRBF, NAF), lambda i: (0, 0)),
            pl.BlockSpec((1, NAF), lambda i: (0, 0)),
        ],
        out_specs=pl.BlockSpec((_FW, 128), lambda i: (i, 0)),
        out_shape=jax.ShapeDtypeStruct((E // 2, 128), jnp.float32),
    )(ea_two, offs, coeff, w, b)


_HB = 1000


def _hupd_body(h_ref, agg_ref, w_ref, b_ref, o_ref):
    z = h_ref[...] + jnp.dot(agg_ref[0], w_ref[...],
                             preferred_element_type=jnp.float32) + b_ref[...]
    o_ref[...] = jax.nn.softplus(z)


def _h_update(h, agg2, w, b):
    return pl.pallas_call(
        _hupd_body,
        grid=(N // _HB,),
        in_specs=[
            pl.BlockSpec((_HB, NAF), lambda i: (i, 0)),
            pl.BlockSpec((1, _HB, NAF),
                         lambda i: (i // (HALF // _HB), i % (HALF // _HB), 0)),
            pl.BlockSpec((NAF, NAF), lambda i: (0, 0)),
            pl.BlockSpec((1, NAF), lambda i: (0, 0)),
        ],
        out_specs=pl.BlockSpec((_HB, NAF), lambda i: (i, 0)),
        out_shape=jax.ShapeDtypeStruct((N, NAF), jnp.float32),
    )(h, agg2, w, b)


def _pool_body(h_ref, batch_ref, u_ref, fcw_ref, fcb_ref,
               hbw1_ref, hbb1_ref, hbw2_ref, hbb2_ref,
               hew1_ref, heb1_ref, hew2_ref, heb2_ref,
               obg_ref, oeh_ref, acc_ref):
    b = pl.program_id(0)

    @pl.when(b == 0)
    def _():
        acc_ref[...] = jnp.zeros((G, 128), jnp.float32)

    gids = lax.broadcasted_iota(jnp.int32, (G, _HB), 0)
    oh = (gids == batch_ref[0, 0, :][None, :]).astype(jnp.float32)
    h_aug = jnp.concatenate(
        [h_ref[...], jnp.ones((_HB, NAF), jnp.float32)], axis=1)
    acc_ref[...] += jnp.dot(oh, h_aug, preferred_element_type=jnp.float32, precision=jax.lax.Precision.HIGHEST)

    @pl.when(b == N // _HB - 1)
    def _():
        sums = acc_ref[:, :NAF]
        cnt = jnp.maximum(acc_ref[:, NAF:NAF + 1], 1.0)
        c = sums / cnt
        gmp = jnp.concatenate([c, u_ref[...]], axis=1)
        z = jax.nn.relu(jnp.dot(gmp, fcw_ref[...],
                                preferred_element_type=jnp.float32)
                        + fcb_ref[...])
        zb = jax.nn.relu(jnp.dot(z, hbw1_ref[...],
                                 preferred_element_type=jnp.float32)
                         + hbb1_ref[...])
        bg = jnp.dot(zb, hbw2_ref[...],
                     preferred_element_type=jnp.float32) + hbb2_ref[...]
        obg_ref[...] = jnp.log1p(jnp.clip(bg, 0.0, None))
        ze = jax.nn.relu(jnp.dot(z, hew1_ref[...],
                                 preferred_element_type=jnp.float32)
                         + heb1_ref[...])
        oeh_ref[...] = jnp.dot(ze, hew2_ref[...],
                               preferred_element_type=jnp.float32) + heb2_ref[...]


def _pool_heads(h, batch3, u, fc_W, fc_b, hb_W1, hb_b1, hb_W2, hb_b2,
                he_W1, he_b1, he_W2, he_b2):
    nb = N // _HB
    full = lambda shape: pl.BlockSpec(shape, lambda i: tuple(0 for _ in shape))
    return pl.pallas_call(
        _pool_body,
        grid=(nb,),
        in_specs=[
            pl.BlockSpec((_HB, NAF), lambda i: (i, 0)),
            pl.BlockSpec((1, 1, _HB), lambda i: (i, 0, 0)),
            full((G, NGF)),
            full((NAF + NGF, 2 * NH)), full((1, 2 * NH)),
            full((2 * NH, NH)), full((1, NH)), full((NH, 1)), full((1, 1)),
            full((2 * NH, NH)), full((1, NH)), full((NH, 1)), full((1, 1)),
        ],
        out_specs=[full((G, 1)), full((G, 1))],
        out_shape=[jax.ShapeDtypeStruct((G, 1), jnp.float32),
                   jax.ShapeDtypeStruct((G, 1), jnp.float32)],
        scratch_shapes=[pltpu.VMEM((G, 128), jnp.float32)],
    )(h, batch3, u, fc_W, fc_b, hb_W1, hb_b1, hb_W2, hb_b2,
      he_W1, he_b1, he_W2, he_b2)


def kernel(x, edge_index, edge_attr, batch, u,
           emb_W, emb_b, b1_W, b1_b, be_W, be_b, b2_W, b2_b,
           fc_W, fc_b, hb_W1, hb_b1, hb_W2, hb_b2,
           he_W1, he_b1, he_W2, he_b2):
    src = edge_index[0]
    dst = edge_index[1]
    ea_two = edge_attr.reshape(E // 2, 2)
    _offs = jnp.linspace(0.0, 6.0, NRBF).reshape(1, NRBF)
    _coeff = (-0.5 / (_offs[0, 1] - _offs[0, 0]) ** 2).reshape(1, 1)

    h = _node_matmul(x, emb_W, emb_b.reshape(1, NAF))
    for i in range(NCONV):
        f = _edge_filter(ea_two, _offs, _coeff, be_W[i],
                         be_b[i].reshape(1, NAF))
        t = _node_matmul(h, b1_W[i], b1_b[i].reshape(1, NAF), pad=True)
        agg2 = _sc_msg_pass_fn()(t, src, dst, f)
        agg2 = agg2.reshape(2, 2 * ACCROWS, NAF)
        h = _h_update(h, agg2, b2_W[i], b2_b[i].reshape(1, NAF))

    out_bg, out_eh = _pool_heads(
        h, batch.reshape(N // _HB, 1, _HB), u,
        fc_W, fc_b.reshape(1, 2 * NH),
        hb_W1, hb_b1.reshape(1, NH), hb_W2, hb_b2.reshape(1, 1),
        he_W1, he_b1.reshape(1, NH), he_W2, he_b2.reshape(1, 1))
    return (out_bg, out_eh)
